# Initial kernel scaffold; baseline (speedup 1.0000x reference)
import jax
import jax.numpy as jnp
from jax import lax
from jax.experimental import pallas as pl
from jax.experimental.pallas import tpu as pltpu
from jax.experimental.pallas import tpu_sc as plsc

N = 10000
D = 128
NC = 2
NS = 16
NW = NC * NS
CHUNK = 128
N_PAD = 10400
RB = 400
TC_BLOCKS = N // RB
TC_BLOCKS_PAD = N_PAD // RB

_vector_mesh = plsc.VectorSubcoreMesh(
    core_axis_name="core", subcore_axis_name="subcore"
)


def _hist_body(src_hbm, dst_hbm, ones_hbm, out_hbm, idx_v, ones_v, tab_s, tab_d):
    c = lax.axis_index("core")
    s = lax.axis_index("subcore")
    w = c * NS + s
    cpw = src_hbm.shape[0] // NW
    rows = tab_s.shape[0] // NS

    pltpu.sync_copy(ones_hbm, ones_v)
    n_z = rows // CHUNK
    @pl.loop(0, n_z)
    def _(r):
        pltpu.sync_copy(ones_hbm.at[pl.ds(CHUNK, CHUNK)],
                        tab_s.at[pl.ds(s * rows + r * CHUNK, CHUNK)])
        pltpu.sync_copy(ones_hbm.at[pl.ds(CHUNK, CHUNK)],
                        tab_d.at[pl.ds(s * rows + r * CHUNK, CHUNK)])
    plsc.subcore_barrier()

    base = w * cpw

    @pl.loop(0, cpw)
    def _(j):
        pltpu.sync_copy(src_hbm.at[base + j], idx_v)
        pltpu.sync_copy(ones_v.at[pl.ds(0, CHUNK)], tab_s.at[idx_v], add=True)

    @pl.loop(0, cpw)
    def _(j):
        pltpu.sync_copy(dst_hbm.at[base + j], idx_v)
        pltpu.sync_copy(ones_v.at[pl.ds(0, CHUNK)], tab_d.at[idx_v], add=True)

    plsc.subcore_barrier()
    pltpu.sync_copy(tab_s.at[pl.ds(s * rows, rows)],
                    out_hbm.at[c, 0, pl.ds(s * rows, rows)])
    pltpu.sync_copy(tab_d.at[pl.ds(s * rows, rows)],
                    out_hbm.at[c, 1, pl.ds(s * rows, rows)])


def _agg_body(h_hbm, src_hbm, dst_hbm, zero_hbm, out_hbm,
              isrc_v, idst_v, vals_v, acc_s, sem):
    c = lax.axis_index("core")
    s = lax.axis_index("subcore")
    w = c * NS + s
    cpw = src_hbm.shape[0] // NW
    rows = N // NS

    pltpu.sync_copy(zero_hbm.at[pl.ds(s * rows, rows)],
                    acc_s.at[pl.ds(s * rows, rows)])
    plsc.subcore_barrier()

    base = w * cpw

    @pl.loop(0, cpw)
    def _(j):
        pltpu.sync_copy(src_hbm.at[base + j], isrc_v)
        pltpu.sync_copy(dst_hbm.at[base + j], idst_v)
        pltpu.async_copy(h_hbm.at[isrc_v], vals_v, sem).wait()
        pltpu.sync_copy(vals_v, acc_s.at[idst_v], add=True)

    plsc.subcore_barrier()
    pltpu.sync_copy(acc_s.at[pl.ds(s * rows, rows)],
                    out_hbm.at[c, pl.ds(s * rows, rows)])


def _sc_hist(src2d, dst2d, ones_z):
    k = pl.kernel(
        _hist_body,
        out_type=jax.ShapeDtypeStruct((NC, 2, N_PAD, 16), jnp.float32),
        mesh=_vector_mesh,
        scratch_types=[
            pltpu.VMEM((CHUNK,), jnp.int32),
            pltpu.VMEM((2 * CHUNK, 16), jnp.float32),
            pltpu.VMEM_SHARED((N_PAD, 16), jnp.float32),
            pltpu.VMEM_SHARED((N_PAD, 16), jnp.float32),
        ],
    )
    return k(src2d, dst2d, ones_z)


def _sc_agg(h_aug, src2d, dst2d, zeros_acc):
    k = pl.kernel(
        _agg_body,
        out_type=jax.ShapeDtypeStruct((NC, N, D), jnp.float32),
        mesh=_vector_mesh,
        scratch_types=[
            pltpu.VMEM((CHUNK,), jnp.int32),
            pltpu.VMEM((CHUNK,), jnp.int32),
            pltpu.VMEM((CHUNK, D), jnp.float32),
            pltpu.VMEM_SHARED((N, D), jnp.float32),
            pltpu.SemaphoreType.DMA,
        ],
    )
    return k(h_aug, src2d, dst2d, zeros_acc)


def _scale_pad_body(x_ref, n_ref, o_ref):
    i = pl.program_id(0)

    @pl.when(i < TC_BLOCKS)
    def _():
        o_ref[...] = x_ref[...] * n_ref[...]

    @pl.when(i >= TC_BLOCKS)
    def _():
        o_ref[...] = jnp.zeros_like(o_ref)


def _layer_mid_body(p0_ref, p1_ref, nd_ref, ns_ref, w_ref, b_ref, o_ref):
    i = pl.program_id(0)

    @pl.when(i < TC_BLOCKS)
    def _():
        a = (p0_ref[...] + p1_ref[...]) * nd_ref[...]
        h = jnp.dot(a, w_ref[...], preferred_element_type=jnp.float32)
        h = h + b_ref[...]
        o_ref[...] = jnp.maximum(h, 0.0) * ns_ref[...]

    @pl.when(i >= TC_BLOCKS)
    def _():
        o_ref[...] = jnp.zeros_like(o_ref)


def _layer_out_body(p0_ref, p1_ref, nd_ref, w_ref, b_ref, o_ref):
    a = (p0_ref[...] + p1_ref[...]) * nd_ref[...]
    h = jnp.dot(a, w_ref[...], preferred_element_type=jnp.float32)
    o_ref[...] = h + b_ref[...]


def _row_block(i):
    return (jnp.minimum(i, TC_BLOCKS - 1), 0)


_ROW_SPEC = pl.BlockSpec((RB, D), _row_block)
_W_SPEC = pl.BlockSpec((D, D), lambda i: (0, 0))
_B_SPEC = pl.BlockSpec((1, D), lambda i: (0, 0))


def _tc_scale_pad(x, n_b):
    return pl.pallas_call(
        _scale_pad_body,
        grid=(TC_BLOCKS_PAD,),
        in_specs=[_ROW_SPEC, _ROW_SPEC],
        out_specs=pl.BlockSpec((RB, D), lambda i: (i, 0)),
        out_shape=jax.ShapeDtypeStruct((N_PAD, D), jnp.float32),
    )(x, n_b)


def _tc_layer_mid(p0, p1, nd_b, ns_b, W, b):
    return pl.pallas_call(
        _layer_mid_body,
        grid=(TC_BLOCKS_PAD,),
        in_specs=[_ROW_SPEC, _ROW_SPEC, _ROW_SPEC, _ROW_SPEC, _W_SPEC, _B_SPEC],
        out_specs=pl.BlockSpec((RB, D), lambda i: (i, 0)),
        out_shape=jax.ShapeDtypeStruct((N_PAD, D), jnp.float32),
    )(p0, p1, nd_b, ns_b, W, b.reshape(1, D))


def _tc_layer_out(p0, p1, nd_b, W, b):
    return pl.pallas_call(
        _layer_out_body,
        grid=(TC_BLOCKS,),
        in_specs=[_ROW_SPEC, _ROW_SPEC, _ROW_SPEC, _W_SPEC, _B_SPEC],
        out_specs=pl.BlockSpec((RB, D), lambda i: (i, 0)),
        out_shape=jax.ShapeDtypeStruct((N, D), jnp.float32),
    )(p0, p1, nd_b, W, b.reshape(1, D))


def kernel(x, edge_index, W1, b1, W2, b2):
    E = edge_index.shape[1]
    cpw = -(-E // (NW * CHUNK))
    e_pad = NW * cpw * CHUNK
    pad = e_pad - E

    src = edge_index[0]
    dst = edge_index[1]
    pad_ids = jnp.arange(pad, dtype=jnp.int32)
    src_p = jnp.concatenate([src, N + pad_ids % (N_PAD - N)])
    dst_main = jnp.concatenate([dst, pad_ids % N])
    dst_hist = jnp.concatenate([dst, N + pad_ids % (N_PAD - N)])
    src2d = src_p.reshape(-1, CHUNK)
    dstm2d = dst_main.reshape(-1, CHUNK)
    dsth2d = dst_hist.reshape(-1, CHUNK)

    ones_z = jnp.concatenate([
        jnp.ones((CHUNK, 16), jnp.float32),
        jnp.zeros((CHUNK, 16), jnp.float32),
    ])
    zeros_acc = jnp.zeros((N, D), jnp.float32)

    tabs = _sc_hist(src2d, dsth2d, ones_z)
    deg_out = tabs[0, 0, :N, 0] + tabs[1, 0, :N, 0]
    deg_in = tabs[0, 1, :N, 0] + tabs[1, 1, :N, 0]
    ns_b = jnp.broadcast_to(
        lax.rsqrt(jnp.maximum(deg_out, 1.0))[:, None], (N, D))
    nd_b = jnp.broadcast_to(
        lax.rsqrt(jnp.maximum(deg_in, 1.0))[:, None], (N, D))

    xs_aug = _tc_scale_pad(x, ns_b)
    p1 = _sc_agg(xs_aug, src2d, dstm2d, zeros_acc)
    h1s = _tc_layer_mid(p1[0], p1[1], nd_b, ns_b, W1, b1)
    p2 = _sc_agg(h1s, src2d, dstm2d, zeros_acc)
    return _tc_layer_out(p2[0], p2[1], nd_b, W2, b2)


# speedup vs baseline: 5.5506x; 5.5506x over previous
import dataclasses

import jax
import jax.numpy as jnp
from jax import lax
from jax.experimental import pallas as pl
from jax.experimental.pallas import tpu as pltpu
from jax.experimental.pallas import tpu_sc as plsc

N = 10000
D = 128
H = 10240
NC = 2
NS = 16
NW = NC * NS
CHUNK = 128
RB = 640
TC_BLOCKS = H // RB

_vector_mesh = plsc.VectorSubcoreMesh(
    core_axis_name="core", subcore_axis_name="subcore"
)

_SC_PARAMS = pltpu.CompilerParams()
if "needs_layout_passes" in pltpu.CompilerParams.__dataclass_fields__:
    _SC_PARAMS = dataclasses.replace(_SC_PARAMS, needs_layout_passes=False)


def _hist_body(src_hbm, dst_hbm, zvec_hbm, out_hbm,
               isrc_v, idst_v, tab_s, tab_d):
    c = lax.axis_index("core")
    s = lax.axis_index("subcore")
    w = c * NS + s
    cpw = src_hbm.shape[0] // NW

    pltpu.sync_copy(zvec_hbm, tab_s)
    pltpu.sync_copy(zvec_hbm, tab_d)
    pltpu.sync_copy(src_hbm.at[pl.ds(w * cpw, cpw)], isrc_v)
    pltpu.sync_copy(dst_hbm.at[pl.ds(w * cpw, cpw)], idst_v)
    ones16 = jnp.full((16,), 1.0, jnp.float32)

    @pl.loop(0, cpw)
    def _(j):
        for k in range(CHUNK // 16):
            iv = isrc_v[j, pl.ds(k * 16, 16)]
            plsc.addupdate_scatter(tab_s, [iv], ones16)
            jv = idst_v[j, pl.ds(k * 16, 16)]
            plsc.addupdate_scatter(tab_d, [jv], ones16)

    pltpu.sync_copy(tab_s, out_hbm.at[c, s, 0])
    pltpu.sync_copy(tab_d, out_hbm.at[c, s, 1])


def _agg_body(h_hbm, src_hbm, dst_hbm, zero_hbm, out_hbm,
              isrc_v, idst_v, vals_v, acc_s, sem):
    c = lax.axis_index("core")
    s = lax.axis_index("subcore")
    w = c * NS + s
    cpw = src_hbm.shape[0] // NW
    rows = H // NS

    pltpu.sync_copy(zero_hbm, acc_s.at[pl.ds(s * rows, rows)])
    plsc.subcore_barrier()

    base = w * cpw

    @pl.loop(0, cpw)
    def _(j):
        pltpu.sync_copy(src_hbm.at[base + j], isrc_v)
        pltpu.sync_copy(dst_hbm.at[base + j], idst_v)
        pltpu.async_copy(h_hbm.at[isrc_v], vals_v, sem).wait()
        pltpu.sync_copy(vals_v, acc_s.at[idst_v], add=True)

    plsc.subcore_barrier()
    pltpu.sync_copy(acc_s.at[pl.ds(s * rows, rows)],
                    out_hbm.at[c, pl.ds(s * rows, rows)])


def _sc_hist(src2d, dst2d, zvec):
    cpw = src2d.shape[0] // NW
    k = pl.kernel(
        _hist_body,
        out_type=jax.ShapeDtypeStruct((NC, NS, 2, H), jnp.float32),
        mesh=_vector_mesh,
        scratch_types=[
            pltpu.VMEM((cpw, CHUNK), jnp.int32),
            pltpu.VMEM((cpw, CHUNK), jnp.int32),
            pltpu.VMEM((H,), jnp.float32),
            pltpu.VMEM((H,), jnp.float32),
        ],
        compiler_params=_SC_PARAMS,
    )
    return k(src2d, dst2d, zvec)


def _sc_agg(h_aug, src2d, dst2d, zrows):
    k = pl.kernel(
        _agg_body,
        out_type=jax.ShapeDtypeStruct((NC, H, D), jnp.float32),
        mesh=_vector_mesh,
        scratch_types=[
            pltpu.VMEM((CHUNK,), jnp.int32),
            pltpu.VMEM((CHUNK,), jnp.int32),
            pltpu.VMEM((CHUNK, D), jnp.float32),
            pltpu.VMEM_SHARED((H, D), jnp.float32),
            pltpu.SemaphoreType.DMA,
        ],
    )
    return k(h_aug, src2d, dst2d, zrows)


def _scale_body(x_ref, n_ref, o_ref):
    o_ref[...] = x_ref[...] * n_ref[...]


def _layer_mid_body(p0_ref, p1_ref, nd_ref, ns_ref, w_ref, b_ref, o_ref):
    a = (p0_ref[...] + p1_ref[...]) * nd_ref[...]
    h = jnp.dot(a, w_ref[...], preferred_element_type=jnp.float32)
    h = h + b_ref[...]
    o_ref[...] = jnp.maximum(h, 0.0) * ns_ref[...]


def _layer_out_body(p0_ref, p1_ref, nd_ref, w_ref, b_ref, o_ref):
    a = (p0_ref[...] + p1_ref[...]) * nd_ref[...]
    h = jnp.dot(a, w_ref[...], preferred_element_type=jnp.float32)
    o_ref[...] = h + b_ref[...]


_ROW_SPEC = pl.BlockSpec((RB, D), lambda i: (i, 0))
_W_SPEC = pl.BlockSpec((D, D), lambda i: (0, 0))
_B_SPEC = pl.BlockSpec((1, D), lambda i: (0, 0))


def _tc_scale(x_pad, n_b):
    return pl.pallas_call(
        _scale_body,
        grid=(TC_BLOCKS,),
        in_specs=[_ROW_SPEC, _ROW_SPEC],
        out_specs=_ROW_SPEC,
        out_shape=jax.ShapeDtypeStruct((H, D), jnp.float32),
    )(x_pad, n_b)


def _tc_layer_mid(p0, p1, nd_b, ns_b, W, b):
    return pl.pallas_call(
        _layer_mid_body,
        grid=(TC_BLOCKS,),
        in_specs=[_ROW_SPEC, _ROW_SPEC, _ROW_SPEC, _ROW_SPEC, _W_SPEC, _B_SPEC],
        out_specs=_ROW_SPEC,
        out_shape=jax.ShapeDtypeStruct((H, D), jnp.float32),
    )(p0, p1, nd_b, ns_b, W, b.reshape(1, D))


def _tc_layer_out(p0, p1, nd_b, W, b):
    return pl.pallas_call(
        _layer_out_body,
        grid=(TC_BLOCKS,),
        in_specs=[_ROW_SPEC, _ROW_SPEC, _ROW_SPEC, _W_SPEC, _B_SPEC],
        out_specs=_ROW_SPEC,
        out_shape=jax.ShapeDtypeStruct((H, D), jnp.float32),
    )(p0, p1, nd_b, W, b.reshape(1, D))


def kernel(x, edge_index, W1, b1, W2, b2):
    E = edge_index.shape[1]
    cpw = -(-E // (NW * CHUNK))
    cpw = -(-cpw // 8) * 8
    e_pad = NW * cpw * CHUNK
    pad = e_pad - E

    src = edge_index[0]
    dst = edge_index[1]
    pad_ids = jnp.arange(pad, dtype=jnp.int32)
    src_p = jnp.concatenate([src, N + pad_ids % (H - N)])
    dst_p = jnp.concatenate([dst, N + pad_ids % (H - N)])
    src2d = src_p.reshape(-1, CHUNK)
    dst2d = dst_p.reshape(-1, CHUNK)

    zvec = jnp.zeros((H,), jnp.float32)
    zrows = jnp.zeros((H // NS, D), jnp.float32)

    tabs = _sc_hist(src2d, dst2d, zvec)
    deg_out = tabs[:, :, 0, :].sum(axis=(0, 1))
    deg_in = tabs[:, :, 1, :].sum(axis=(0, 1))
    row_valid = jnp.arange(H) < N
    ns = jnp.where(row_valid, lax.rsqrt(jnp.maximum(deg_out, 1.0)), 0.0)
    nd = jnp.where(row_valid, lax.rsqrt(jnp.maximum(deg_in, 1.0)), 0.0)
    ns_b = jnp.broadcast_to(ns[:, None], (H, D))
    nd_b = jnp.broadcast_to(nd[:, None], (H, D))

    x_pad = jnp.zeros((H, D), jnp.float32).at[:N].set(x)
    xs_aug = _tc_scale(x_pad, ns_b)
    p1 = _sc_agg(xs_aug, src2d, dst2d, zrows)
    h1s = _tc_layer_mid(p1[0], p1[1], nd_b, ns_b, W1, b1)
    p2 = _sc_agg(h1s, src2d, dst2d, zrows)
    out = _tc_layer_out(p2[0], p2[1], nd_b, W2, b2)
    return out[:N]


# speedup vs baseline: 9.3850x; 1.6908x over previous
import dataclasses

import jax
import jax.numpy as jnp
from jax import lax
from jax.experimental import pallas as pl
from jax.experimental.pallas import tpu as pltpu
from jax.experimental.pallas import tpu_sc as plsc

N = 10000
D = 128
H = 10240
NC = 2
NS = 16
NW = NC * NS
CHUNK = 128
RB = 640
TC_BLOCKS = H // RB

_vector_mesh = plsc.VectorSubcoreMesh(
    core_axis_name="core", subcore_axis_name="subcore"
)

_SC_PARAMS = pltpu.CompilerParams()
if "needs_layout_passes" in pltpu.CompilerParams.__dataclass_fields__:
    _SC_PARAMS = dataclasses.replace(_SC_PARAMS, needs_layout_passes=False)


def _hist_body(src_hbm, dst_hbm, zvec_hbm, out_hbm,
               isrc_v, idst_v, tab_s, tab_d):
    c = lax.axis_index("core")
    s = lax.axis_index("subcore")
    w = c * NS + s
    cpw = src_hbm.shape[0] // NW

    pltpu.sync_copy(zvec_hbm, tab_s)
    pltpu.sync_copy(zvec_hbm, tab_d)
    pltpu.sync_copy(src_hbm.at[pl.ds(w * cpw, cpw)], isrc_v)
    pltpu.sync_copy(dst_hbm.at[pl.ds(w * cpw, cpw)], idst_v)
    ones16 = jnp.full((16,), 1.0, jnp.float32)

    @pl.loop(0, cpw)
    def _(j):
        for k in range(CHUNK // 16):
            iv = isrc_v[j, pl.ds(k * 16, 16)]
            plsc.addupdate_scatter(tab_s, [iv], ones16)
            jv = idst_v[j, pl.ds(k * 16, 16)]
            plsc.addupdate_scatter(tab_d, [jv], ones16)

    pltpu.sync_copy(tab_s, out_hbm.at[c, s, 0])
    pltpu.sync_copy(tab_d, out_hbm.at[c, s, 1])


BLK = 8


def _agg_body(h_hbm, src_hbm, dst_hbm, zero_hbm, out_hbm,
              isa, ida, isb, idb, vals0, vals1, acc_s,
              sem0, sem1, sia, sib):
    c = lax.axis_index("core")
    s = lax.axis_index("subcore")
    w = c * NS + s
    cpw = src_hbm.shape[0] // NW
    nblk = cpw // BLK
    rows = H // NS
    base = w * cpw

    pltpu.sync_copy(zero_hbm, acc_s.at[pl.ds(s * rows, rows)])
    plsc.subcore_barrier()

    def load_idx(blk, is_ref, id_ref, sem):
        off = base + blk * BLK
        pltpu.async_copy(src_hbm.at[pl.ds(off, BLK)], is_ref, sem)
        pltpu.async_copy(dst_hbm.at[pl.ds(off, BLK)], id_ref, sem)

    def wait_idx(blk, is_ref, id_ref, sem):
        off = base + blk * BLK
        pltpu.make_async_copy(src_hbm.at[pl.ds(off, BLK)], is_ref, sem).wait()
        pltpu.make_async_copy(dst_hbm.at[pl.ds(off, BLK)], id_ref, sem).wait()

    def run_block(ib, jb):
        pltpu.async_copy(h_hbm.at[ib.at[0]], vals0, sem0)
        pltpu.async_copy(h_hbm.at[ib.at[1]], vals1, sem1)
        for k in range(0, BLK, 2):
            pltpu.make_async_copy(h_hbm.at[ib.at[k]], vals0, sem0).wait()
            pltpu.sync_copy(vals0, acc_s.at[jb.at[k]], add=True)
            if k + 2 < BLK:
                pltpu.async_copy(h_hbm.at[ib.at[k + 2]], vals0, sem0)
            pltpu.make_async_copy(h_hbm.at[ib.at[k + 1]], vals1, sem1).wait()
            pltpu.sync_copy(vals1, acc_s.at[jb.at[k + 1]], add=True)
            if k + 3 < BLK:
                pltpu.async_copy(h_hbm.at[ib.at[k + 3]], vals1, sem1)

    load_idx(0, isa, ida, sia)

    @pl.loop(0, nblk // 2)
    def _(i):
        blk = 2 * i
        wait_idx(blk, isa, ida, sia)
        load_idx(blk + 1, isb, idb, sib)
        run_block(isa, ida)
        wait_idx(blk + 1, isb, idb, sib)

        @pl.when(blk + 2 < nblk)
        def _():
            load_idx(blk + 2, isa, ida, sia)

        run_block(isb, idb)

    plsc.subcore_barrier()
    pltpu.sync_copy(acc_s.at[pl.ds(s * rows, rows)],
                    out_hbm.at[c, pl.ds(s * rows, rows)])


def _sc_hist(src2d, dst2d, zvec):
    cpw = src2d.shape[0] // NW
    k = pl.kernel(
        _hist_body,
        out_type=jax.ShapeDtypeStruct((NC, NS, 2, H), jnp.float32),
        mesh=_vector_mesh,
        scratch_types=[
            pltpu.VMEM((cpw, CHUNK), jnp.int32),
            pltpu.VMEM((cpw, CHUNK), jnp.int32),
            pltpu.VMEM((H,), jnp.float32),
            pltpu.VMEM((H,), jnp.float32),
        ],
        compiler_params=_SC_PARAMS,
    )
    return k(src2d, dst2d, zvec)


def _sc_agg(h_aug, src2d, dst2d, zrows):
    cpw = src2d.shape[0] // NW
    k = pl.kernel(
        _agg_body,
        out_type=jax.ShapeDtypeStruct((NC, H, D), jnp.float32),
        mesh=_vector_mesh,
        scratch_types=[
            pltpu.VMEM((BLK, CHUNK), jnp.int32),
            pltpu.VMEM((BLK, CHUNK), jnp.int32),
            pltpu.VMEM((BLK, CHUNK), jnp.int32),
            pltpu.VMEM((BLK, CHUNK), jnp.int32),
            pltpu.VMEM((CHUNK, D), jnp.float32),
            pltpu.VMEM((CHUNK, D), jnp.float32),
            pltpu.VMEM_SHARED((H, D), jnp.float32),
            pltpu.SemaphoreType.DMA,
            pltpu.SemaphoreType.DMA,
            pltpu.SemaphoreType.DMA,
            pltpu.SemaphoreType.DMA,
        ],
    )
    return k(h_aug, src2d, dst2d, zrows)


def _scale_body(x_ref, n_ref, o_ref):
    o_ref[...] = x_ref[...] * n_ref[...]


def _layer_mid_body(p0_ref, p1_ref, nd_ref, ns_ref, w_ref, b_ref, o_ref):
    a = (p0_ref[...] + p1_ref[...]) * nd_ref[...]
    h = jnp.dot(a, w_ref[...], preferred_element_type=jnp.float32)
    h = h + b_ref[...]
    o_ref[...] = jnp.maximum(h, 0.0) * ns_ref[...]


def _layer_out_body(p0_ref, p1_ref, nd_ref, w_ref, b_ref, o_ref):
    a = (p0_ref[...] + p1_ref[...]) * nd_ref[...]
    h = jnp.dot(a, w_ref[...], preferred_element_type=jnp.float32)
    o_ref[...] = h + b_ref[...]


_ROW_SPEC = pl.BlockSpec((RB, D), lambda i: (i, 0))
_W_SPEC = pl.BlockSpec((D, D), lambda i: (0, 0))
_B_SPEC = pl.BlockSpec((1, D), lambda i: (0, 0))


def _tc_scale(x_pad, n_b):
    return pl.pallas_call(
        _scale_body,
        grid=(TC_BLOCKS,),
        in_specs=[_ROW_SPEC, _ROW_SPEC],
        out_specs=_ROW_SPEC,
        out_shape=jax.ShapeDtypeStruct((H, D), jnp.float32),
    )(x_pad, n_b)


def _tc_layer_mid(p0, p1, nd_b, ns_b, W, b):
    return pl.pallas_call(
        _layer_mid_body,
        grid=(TC_BLOCKS,),
        in_specs=[_ROW_SPEC, _ROW_SPEC, _ROW_SPEC, _ROW_SPEC, _W_SPEC, _B_SPEC],
        out_specs=_ROW_SPEC,
        out_shape=jax.ShapeDtypeStruct((H, D), jnp.float32),
    )(p0, p1, nd_b, ns_b, W, b.reshape(1, D))


def _tc_layer_out(p0, p1, nd_b, W, b):
    return pl.pallas_call(
        _layer_out_body,
        grid=(TC_BLOCKS,),
        in_specs=[_ROW_SPEC, _ROW_SPEC, _ROW_SPEC, _W_SPEC, _B_SPEC],
        out_specs=_ROW_SPEC,
        out_shape=jax.ShapeDtypeStruct((H, D), jnp.float32),
    )(p0, p1, nd_b, W, b.reshape(1, D))


def kernel(x, edge_index, W1, b1, W2, b2):
    E = edge_index.shape[1]
    cpw = -(-E // (NW * CHUNK))
    cpw = -(-cpw // (2 * BLK)) * (2 * BLK)
    e_pad = NW * cpw * CHUNK
    pad = e_pad - E

    src = edge_index[0]
    dst = edge_index[1]
    pad_ids = jnp.arange(pad, dtype=jnp.int32)
    src_p = jnp.concatenate([src, N + pad_ids % (H - N)])
    dst_p = jnp.concatenate([dst, N + pad_ids % (H - N)])
    src2d = src_p.reshape(-1, CHUNK)
    dst2d = dst_p.reshape(-1, CHUNK)

    zvec = jnp.zeros((H,), jnp.float32)
    zrows = jnp.zeros((H // NS, D), jnp.float32)

    tabs = _sc_hist(src2d, dst2d, zvec)
    deg_out = tabs[:, :, 0, :].sum(axis=(0, 1))
    deg_in = tabs[:, :, 1, :].sum(axis=(0, 1))
    row_valid = jnp.arange(H) < N
    ns = jnp.where(row_valid, lax.rsqrt(jnp.maximum(deg_out, 1.0)), 0.0)
    nd = jnp.where(row_valid, lax.rsqrt(jnp.maximum(deg_in, 1.0)), 0.0)
    ns_b = jnp.broadcast_to(ns[:, None], (H, D))
    nd_b = jnp.broadcast_to(nd[:, None], (H, D))

    x_pad = jnp.zeros((H, D), jnp.float32).at[:N].set(x)
    xs_aug = _tc_scale(x_pad, ns_b)
    p1 = _sc_agg(xs_aug, src2d, dst2d, zrows)
    h1s = _tc_layer_mid(p1[0], p1[1], nd_b, ns_b, W1, b1)
    p2 = _sc_agg(h1s, src2d, dst2d, zrows)
    out = _tc_layer_out(p2[0], p2[1], nd_b, W2, b2)
    return out[:N]
